# Initial kernel scaffold; baseline (speedup 1.0000x reference)
import jax
import jax.numpy as jnp
from jax.experimental import pallas as pl


def kernel(x, node_attr, edge_attr, edge_index, graph_idx, W_embed, W_msg0, W_msg1, W_upd0, W_upd1, W_pre0, W_pre1, W_post0, W_out):
    raise NotImplementedError("write your pallas kernel here")


# speedup vs baseline: 1.5597x; 1.5597x over previous
import functools
import math

import jax
import jax.numpy as jnp
from jax import lax
from jax.experimental import pallas as pl
from jax.experimental.pallas import tpu as pltpu
from jax.experimental.pallas import tpu_sc as plsc

D = 128
A = 4
G = 16
_CHUNK = 128


def _silu(v):
    return v * jax.nn.sigmoid(v)


def _contract(y, ea, scale):
    acc = ea[:, 0:1] * y[:, 0:D]
    for j in range(1, A):
        acc = acc + ea[:, j:j + 1] * y[:, j * D:(j + 1) * D]
    return acc * scale


def _sc_counts():
    try:
        info = plsc.get_sparse_core_info()
        return int(info.num_cores), int(info.num_subcores)
    except Exception:
        return 2, 16


def _embed_body(x_ref, a_ref, w_ref, o_ref):
    y = jnp.dot(x_ref[...], w_ref[...], preferred_element_type=jnp.float32)
    o_ref[...] = _contract(y, a_ref[...], 1.0 / math.sqrt(D * A))


def _embed(x, nattr, w_r):
    n = x.shape[0]
    bn = 1000
    return pl.pallas_call(
        _embed_body,
        grid=(n // bn,),
        in_specs=[
            pl.BlockSpec((bn, D), lambda i: (i, 0)),
            pl.BlockSpec((bn, A), lambda i: (i, 0)),
            pl.BlockSpec((D, A * D), lambda i: (0, 0)),
        ],
        out_specs=pl.BlockSpec((bn, D), lambda i: (i, 0)),
        out_shape=jax.ShapeDtypeStruct((n, D), jnp.float32),
    )(x, nattr, w_r)


def _edge_body(inc_ref, outg_ref, ea_ref, w0a_ref, w0b_ref, w1_ref, o_ref):
    ea = ea_ref[...]
    y0 = jnp.dot(inc_ref[...], w0a_ref[...], preferred_element_type=jnp.float32)
    y0 = y0 + jnp.dot(outg_ref[...], w0b_ref[...],
                      preferred_element_type=jnp.float32)
    m = _silu(_contract(y0, ea, 1.0 / math.sqrt(2 * D * A)))
    y1 = jnp.dot(m, w1_ref[...], preferred_element_type=jnp.float32)
    o_ref[...] = _silu(_contract(y1, ea, 1.0 / math.sqrt(D * A)))


def _edge_mlp(inc, outg, ea, w0a, w0b, w1):
    ep = inc.shape[0]
    be = 2048
    return pl.pallas_call(
        _edge_body,
        grid=(ep // be,),
        in_specs=[
            pl.BlockSpec((be, D), lambda i: (i, 0)),
            pl.BlockSpec((be, D), lambda i: (i, 0)),
            pl.BlockSpec((be, A), lambda i: (i, 0)),
            pl.BlockSpec((D, A * D), lambda i: (0, 0)),
            pl.BlockSpec((D, A * D), lambda i: (0, 0)),
            pl.BlockSpec((D, A * D), lambda i: (0, 0)),
        ],
        out_specs=pl.BlockSpec((be, D), lambda i: (i, 0)),
        out_shape=jax.ShapeDtypeStruct((ep, D), jnp.float32),
    )(inc, outg, ea, w0a, w0b, w1)


def _update_body(nd_ref, a0_ref, a1_ref, na_ref, w0a_ref, w0b_ref, w1_ref,
                 o_ref):
    nd = nd_ref[...]
    agg = a0_ref[0] + a1_ref[0]
    na = na_ref[...]
    y0 = jnp.dot(nd, w0a_ref[...], preferred_element_type=jnp.float32)
    y0 = y0 + jnp.dot(agg, w0b_ref[...], preferred_element_type=jnp.float32)
    u = _silu(_contract(y0, na, 1.0 / math.sqrt(2 * D * A)))
    y1 = jnp.dot(u, w1_ref[...], preferred_element_type=jnp.float32)
    o_ref[...] = nd + _contract(y1, na, 1.0 / math.sqrt(D * A))


def _update(nodes, agg, nattr, w0a, w0b, w1):
    n = nodes.shape[0]
    bn = 1000
    return pl.pallas_call(
        _update_body,
        grid=(n // bn,),
        in_specs=[
            pl.BlockSpec((bn, D), lambda i: (i, 0)),
            pl.BlockSpec((1, bn, D), lambda i: (0, i, 0)),
            pl.BlockSpec((1, bn, D), lambda i: (1, i, 0)),
            pl.BlockSpec((bn, A), lambda i: (i, 0)),
            pl.BlockSpec((D, A * D), lambda i: (0, 0)),
            pl.BlockSpec((D, A * D), lambda i: (0, 0)),
            pl.BlockSpec((D, A * D), lambda i: (0, 0)),
        ],
        out_specs=pl.BlockSpec((bn, D), lambda i: (i, 0)),
        out_shape=jax.ShapeDtypeStruct((n, D), jnp.float32),
    )(nodes, agg, agg, nattr, w0a, w0b, w1)


def _prepool_body(nd_ref, na_ref, w0_ref, w1_ref, o_ref):
    na = na_ref[...]
    y0 = jnp.dot(nd_ref[...], w0_ref[...], preferred_element_type=jnp.float32)
    h = _silu(_contract(y0, na, 1.0 / math.sqrt(D * A)))
    y1 = jnp.dot(h, w1_ref[...], preferred_element_type=jnp.float32)
    o_ref[...] = _contract(y1, na, 1.0 / math.sqrt(D * A))


def _prepool(nodes, nattr, w0, w1):
    n = nodes.shape[0]
    bn = 1000
    return pl.pallas_call(
        _prepool_body,
        grid=(n // bn,),
        in_specs=[
            pl.BlockSpec((bn, D), lambda i: (i, 0)),
            pl.BlockSpec((bn, A), lambda i: (i, 0)),
            pl.BlockSpec((D, A * D), lambda i: (0, 0)),
            pl.BlockSpec((D, A * D), lambda i: (0, 0)),
        ],
        out_specs=pl.BlockSpec((bn, D), lambda i: (i, 0)),
        out_shape=jax.ShapeDtypeStruct((n, D), jnp.float32),
    )(nodes, nattr, w0, w1)


def _pool_body(h_ref, gi_ref, wpost_ref, wout_ref, o_ref, sums, cnt):
    i = pl.program_id(0)

    @pl.when(i == 0)
    def _():
        sums[...] = jnp.zeros_like(sums)
        cnt[...] = jnp.zeros_like(cnt)

    gi = gi_ref[...]
    bn = gi.shape[0]
    m = (gi == lax.broadcasted_iota(jnp.int32, (bn, G), 1)).astype(jnp.float32)
    h = h_ref[...]
    dn = (((0,), (0,)), ((), ()))
    sums[...] += lax.dot_general(m, h, dn, preferred_element_type=jnp.float32)
    cnt[...] += lax.dot_general(m, jnp.ones_like(h), dn,
                                preferred_element_type=jnp.float32)
    pooled = sums[...] / jnp.maximum(cnt[...], 1.0)
    h2 = _silu(jnp.dot(pooled, wpost_ref[...],
                       preferred_element_type=jnp.float32) / math.sqrt(D))
    o_ref[...] = jnp.dot(h2, wout_ref[...],
                         preferred_element_type=jnp.float32) / math.sqrt(D)


def _pool_decode(h, gi2d, wpost, wout):
    n = h.shape[0]
    bn = 1000
    return pl.pallas_call(
        _pool_body,
        grid=(n // bn,),
        in_specs=[
            pl.BlockSpec((bn, D), lambda i: (i, 0)),
            pl.BlockSpec((bn, 1), lambda i: (i, 0)),
            pl.BlockSpec((D, D), lambda i: (0, 0)),
            pl.BlockSpec((D, 1), lambda i: (0, 0)),
        ],
        out_specs=pl.BlockSpec((G, 1), lambda i: (0, 0)),
        out_shape=jax.ShapeDtypeStruct((G, 1), jnp.float32),
        scratch_shapes=[
            pltpu.VMEM((G, D), jnp.float32),
            pltpu.VMEM((G, D), jnp.float32),
        ],
    )(h, gi2d, wpost, wout)


def _sc_gather(nodes, s_idx, r_idx):
    nc, ns = _sc_counts()
    nw = nc * ns
    ep = s_idx.shape[0]
    per_w = ep // nw
    n_ch = per_w // _CHUNK
    mesh = plsc.VectorSubcoreMesh(core_axis_name="c", subcore_axis_name="s")
    out_t = (jax.ShapeDtypeStruct((ep, D), jnp.float32),
             jax.ShapeDtypeStruct((ep, D), jnp.float32))

    @functools.partial(
        pl.kernel, mesh=mesh, out_type=out_t,
        scratch_types=[
            pltpu.VMEM((_CHUNK,), jnp.int32),
            pltpu.VMEM((_CHUNK,), jnp.int32),
            pltpu.VMEM((_CHUNK, D), jnp.float32),
            pltpu.VMEM((_CHUNK, D), jnp.float32),
            pltpu.SemaphoreType.DMA,
            pltpu.SemaphoreType.DMA,
        ],
    )
    def k(nodes_h, s_h, r_h, inc_h, outg_h, ix_s, ix_r, rw_s, rw_r, sem_s,
          sem_r):
        wid = lax.axis_index("s") * nc + lax.axis_index("c")
        base = wid * per_w

        def body(i, _):
            off = base + i * _CHUNK
            pltpu.sync_copy(s_h.at[pl.ds(off, _CHUNK)], ix_s)
            pltpu.sync_copy(r_h.at[pl.ds(off, _CHUNK)], ix_r)
            a = pltpu.async_copy(nodes_h.at[ix_s], rw_s, sem_s)
            b = pltpu.async_copy(nodes_h.at[ix_r], rw_r, sem_r)
            a.wait()
            pltpu.sync_copy(rw_s, inc_h.at[pl.ds(off, _CHUNK)])
            b.wait()
            pltpu.sync_copy(rw_r, outg_h.at[pl.ds(off, _CHUNK)])
            return 0

        lax.fori_loop(0, n_ch, body, 0, unroll=False)

    return k(nodes, s_idx, r_idx)


def _sc_scatter(msg, r_idx, nrow):
    nc, ns = _sc_counts()
    nw = nc * ns
    ep = msg.shape[0]
    per_w = ep // nw
    n_ch = per_w // _CHUNK
    rows_t = nrow // ns
    mesh = plsc.VectorSubcoreMesh(core_axis_name="c", subcore_axis_name="s")
    out_t = jax.ShapeDtypeStruct((nc, nrow, D), jnp.float32)

    chunks = []
    off = 0
    while off < rows_t:
        sz = min(_CHUNK, rows_t - off)
        chunks.append((off, sz))
        off += sz

    @functools.partial(
        pl.kernel, mesh=mesh, out_type=out_t,
        scratch_types=[
            pltpu.VMEM((_CHUNK,), jnp.int32),
            pltpu.VMEM((_CHUNK, D), jnp.float32),
            pltpu.VMEM_SHARED((nrow, D), jnp.float32),
            pltpu.SemaphoreType.DMA,
        ],
    )
    def k(msg_h, r_h, out_h, ix, rw, acc, sem):
        cid = lax.axis_index("c")
        sid = lax.axis_index("s")
        wid = sid * nc + cid
        base = wid * per_w
        row0 = sid * rows_t

        def zr(r, _):
            def zc(c, __):
                rw[r, pl.ds(c * 16, 16)] = jnp.zeros((16,), jnp.float32)
                return 0
            lax.fori_loop(0, D // 16, zc, 0, unroll=True)
            return 0

        lax.fori_loop(0, _CHUNK, zr, 0, unroll=False)
        for coff, csz in chunks:
            pltpu.sync_copy(rw.at[pl.ds(0, csz)],
                            acc.at[pl.ds(row0 + coff, csz)])
        plsc.subcore_barrier()

        def body(i, _):
            off = base + i * _CHUNK
            pltpu.sync_copy(r_h.at[pl.ds(off, _CHUNK)], ix)
            pltpu.sync_copy(msg_h.at[pl.ds(off, _CHUNK)], rw)
            pltpu.sync_copy(rw, acc.at[ix], add=True)
            return 0

        lax.fori_loop(0, n_ch, body, 0, unroll=False)
        plsc.subcore_barrier()

        for coff, csz in chunks:
            pltpu.sync_copy(acc.at[pl.ds(row0 + coff, csz)],
                            rw.at[pl.ds(0, csz)])
            pltpu.sync_copy(rw.at[pl.ds(0, csz)],
                            out_h.at[cid, pl.ds(row0 + coff, csz)])

    return k(msg, r_idx)


def kernel(x, node_attr, edge_attr, edge_index, graph_idx, W_embed, W_msg0,
           W_msg1, W_upd0, W_upd1, W_pre0, W_pre1, W_post0, W_out):
    n, d = x.shape
    e = edge_index.shape[1]
    nc, ns = _sc_counts()
    nw = nc * ns
    quant = nw * _CHUNK
    ep = ((e + quant - 1) // quant) * quant
    pad = ep - e
    nrow = ((n + 1 + 127) // 128) * 128

    senders = edge_index[0].astype(jnp.int32)
    receivers = edge_index[1].astype(jnp.int32)
    zpad = jnp.zeros((pad,), jnp.int32)
    s_p = jnp.concatenate([senders, zpad])
    r_p = jnp.concatenate([receivers, zpad])
    r_scat = jnp.concatenate([receivers, jnp.full((pad,), n, jnp.int32)])
    ea_p = jnp.concatenate(
        [edge_attr, jnp.zeros((pad, A), jnp.float32)], axis=0)

    w_embed_r = W_embed.reshape(D, A * D)
    nodes = _embed(x, node_attr, w_embed_r)

    num_layers = W_msg0.shape[0]
    for l in range(num_layers):
        w0a = W_msg0[l, :D].reshape(D, A * D)
        w0b = W_msg0[l, D:].reshape(D, A * D)
        w1 = W_msg1[l].reshape(D, A * D)
        u0a = W_upd0[l, :D].reshape(D, A * D)
        u0b = W_upd0[l, D:].reshape(D, A * D)
        u1 = W_upd1[l].reshape(D, A * D)

        inc, outg = _sc_gather(nodes, s_p, r_p)
        msg = _edge_mlp(inc, outg, ea_p, w0a, w0b, w1)
        agg = _sc_scatter(msg, r_scat, nrow)
        nodes = _update(nodes, agg, node_attr, u0a, u0b, u1)

    h = _prepool(nodes, node_attr, W_pre0.reshape(D, A * D),
                 W_pre1.reshape(D, A * D))
    gi2d = graph_idx.astype(jnp.int32).reshape(n, 1)
    out = _pool_decode(h, gi2d, W_post0, W_out)
    return out.reshape(G)


# speedup vs baseline: 3.0504x; 1.9557x over previous
import functools
import math

import jax
import jax.numpy as jnp
from jax import lax
from jax.experimental import pallas as pl
from jax.experimental.pallas import tpu as pltpu
from jax.experimental.pallas import tpu_sc as plsc

D = 128
A = 4
G = 16
_CHUNK = 128


def _silu(v):
    return v * jax.nn.sigmoid(v)


def _contract(y, ea, scale):
    acc = ea[:, 0:1] * y[:, 0:D]
    for j in range(1, A):
        acc = acc + ea[:, j:j + 1] * y[:, j * D:(j + 1) * D]
    return acc * scale


def _sc_counts():
    try:
        info = plsc.get_sparse_core_info()
        return int(info.num_cores), int(info.num_subcores)
    except Exception:
        return 2, 16


def _embed_body(x_ref, a_ref, w_ref, o_ref):
    y = jnp.dot(x_ref[...], w_ref[...], preferred_element_type=jnp.float32)
    o_ref[...] = _contract(y, a_ref[...], 1.0 / math.sqrt(D * A))


def _embed(x, nattr, w_r):
    n = x.shape[0]
    bn = n // 16
    return pl.pallas_call(
        _embed_body,
        grid=(n // bn,),
        in_specs=[
            pl.BlockSpec((bn, D), lambda i: (i, 0)),
            pl.BlockSpec((bn, A), lambda i: (i, 0)),
            pl.BlockSpec((D, A * D), lambda i: (0, 0)),
        ],
        out_specs=pl.BlockSpec((bn, D), lambda i: (i, 0)),
        out_shape=jax.ShapeDtypeStruct((n, D), jnp.float32),
    )(x, nattr, w_r)


def _edge_body(inc_ref, outg_ref, ea_ref, w0a_ref, w0b_ref, w1_ref, o_ref):
    ea = ea_ref[...]
    y0 = jnp.dot(inc_ref[...], w0a_ref[...], preferred_element_type=jnp.float32)
    y0 = y0 + jnp.dot(outg_ref[...], w0b_ref[...],
                      preferred_element_type=jnp.float32)
    m = _silu(_contract(y0, ea, 1.0 / math.sqrt(2 * D * A)))
    y1 = jnp.dot(m, w1_ref[...], preferred_element_type=jnp.float32)
    o_ref[...] = _silu(_contract(y1, ea, 1.0 / math.sqrt(D * A)))


def _edge_mlp(inc, outg, ea, w0a, w0b, w1):
    ep = inc.shape[0]
    be = 2048
    return pl.pallas_call(
        _edge_body,
        grid=(ep // be,),
        in_specs=[
            pl.BlockSpec((be, D), lambda i: (i, 0)),
            pl.BlockSpec((be, D), lambda i: (i, 0)),
            pl.BlockSpec((be, A), lambda i: (i, 0)),
            pl.BlockSpec((D, A * D), lambda i: (0, 0)),
            pl.BlockSpec((D, A * D), lambda i: (0, 0)),
            pl.BlockSpec((D, A * D), lambda i: (0, 0)),
        ],
        out_specs=pl.BlockSpec((be, D), lambda i: (i, 0)),
        out_shape=jax.ShapeDtypeStruct((ep, D), jnp.float32),
    )(inc, outg, ea, w0a, w0b, w1)


def _update_body(nd_ref, a0_ref, a1_ref, na_ref, w0a_ref, w0b_ref, w1_ref,
                 o_ref):
    nd = nd_ref[...]
    agg = a0_ref[0] + a1_ref[0]
    na = na_ref[...]
    y0 = jnp.dot(nd, w0a_ref[...], preferred_element_type=jnp.float32)
    y0 = y0 + jnp.dot(agg, w0b_ref[...], preferred_element_type=jnp.float32)
    u = _silu(_contract(y0, na, 1.0 / math.sqrt(2 * D * A)))
    y1 = jnp.dot(u, w1_ref[...], preferred_element_type=jnp.float32)
    o_ref[...] = nd + _contract(y1, na, 1.0 / math.sqrt(D * A))


def _update(nodes, agg, nattr, w0a, w0b, w1):
    n = nodes.shape[0]
    bn = n // 16
    return pl.pallas_call(
        _update_body,
        grid=(n // bn,),
        in_specs=[
            pl.BlockSpec((bn, D), lambda i: (i, 0)),
            pl.BlockSpec((1, bn, D), lambda i: (0, i, 0)),
            pl.BlockSpec((1, bn, D), lambda i: (1, i, 0)),
            pl.BlockSpec((bn, A), lambda i: (i, 0)),
            pl.BlockSpec((D, A * D), lambda i: (0, 0)),
            pl.BlockSpec((D, A * D), lambda i: (0, 0)),
            pl.BlockSpec((D, A * D), lambda i: (0, 0)),
        ],
        out_specs=pl.BlockSpec((bn, D), lambda i: (i, 0)),
        out_shape=jax.ShapeDtypeStruct((n, D), jnp.float32),
    )(nodes, agg, agg, nattr, w0a, w0b, w1)


def _prepool_body(nd_ref, na_ref, w0_ref, w1_ref, o_ref):
    na = na_ref[...]
    y0 = jnp.dot(nd_ref[...], w0_ref[...], preferred_element_type=jnp.float32)
    h = _silu(_contract(y0, na, 1.0 / math.sqrt(D * A)))
    y1 = jnp.dot(h, w1_ref[...], preferred_element_type=jnp.float32)
    o_ref[...] = _contract(y1, na, 1.0 / math.sqrt(D * A))


def _prepool(nodes, nattr, w0, w1):
    n = nodes.shape[0]
    bn = n // 16
    return pl.pallas_call(
        _prepool_body,
        grid=(n // bn,),
        in_specs=[
            pl.BlockSpec((bn, D), lambda i: (i, 0)),
            pl.BlockSpec((bn, A), lambda i: (i, 0)),
            pl.BlockSpec((D, A * D), lambda i: (0, 0)),
            pl.BlockSpec((D, A * D), lambda i: (0, 0)),
        ],
        out_specs=pl.BlockSpec((bn, D), lambda i: (i, 0)),
        out_shape=jax.ShapeDtypeStruct((n, D), jnp.float32),
    )(nodes, nattr, w0, w1)


def _pool_body(h_ref, gi_ref, wpost_ref, wout_ref, o_ref, sums, cnt):
    i = pl.program_id(0)

    @pl.when(i == 0)
    def _():
        sums[...] = jnp.zeros_like(sums)
        cnt[...] = jnp.zeros_like(cnt)

    gi = gi_ref[...]
    bn = gi.shape[0]
    m = (gi == lax.broadcasted_iota(jnp.int32, (bn, G), 1)).astype(jnp.float32)
    h = h_ref[...]
    dn = (((0,), (0,)), ((), ()))
    sums[...] += lax.dot_general(m, h, dn, preferred_element_type=jnp.float32)
    cnt[...] += lax.dot_general(m, jnp.ones_like(h), dn,
                                preferred_element_type=jnp.float32)
    pooled = sums[...] / jnp.maximum(cnt[...], 1.0)
    h2 = _silu(jnp.dot(pooled, wpost_ref[...],
                       preferred_element_type=jnp.float32) / math.sqrt(D))
    o_ref[...] = jnp.dot(h2, wout_ref[...],
                         preferred_element_type=jnp.float32) / math.sqrt(D)


def _pool_decode(h, gi2d, wpost, wout):
    n = h.shape[0]
    bn = n // 16
    return pl.pallas_call(
        _pool_body,
        grid=(n // bn,),
        in_specs=[
            pl.BlockSpec((bn, D), lambda i: (i, 0)),
            pl.BlockSpec((bn, 1), lambda i: (i, 0)),
            pl.BlockSpec((D, D), lambda i: (0, 0)),
            pl.BlockSpec((D, 1), lambda i: (0, 0)),
        ],
        out_specs=pl.BlockSpec((G, 1), lambda i: (0, 0)),
        out_shape=jax.ShapeDtypeStruct((G, 1), jnp.float32),
        scratch_shapes=[
            pltpu.VMEM((G, D), jnp.float32),
            pltpu.VMEM((G, D), jnp.float32),
        ],
    )(h, gi2d, wpost, wout)


def _chunks_of(total, cap):
    out, off = [], 0
    while off < total:
        sz = min(cap, total - off)
        out.append((off, sz))
        off += sz
    return out


def _sc_gather(nodes, s_idx, r_idx):
    nc, ns = _sc_counts()
    nw = nc * ns
    ep = s_idx.shape[0]
    npad = nodes.shape[0]
    rt = npad // ns
    per_w = ep // nw
    n_ch = per_w // _CHUNK
    stage_chunks = _chunks_of(rt, _CHUNK)
    mesh = plsc.VectorSubcoreMesh(core_axis_name="c", subcore_axis_name="s")
    out_t = (jax.ShapeDtypeStruct((ep, D), jnp.float32),
             jax.ShapeDtypeStruct((ep, D), jnp.float32))

    @functools.partial(
        pl.kernel, mesh=mesh, out_type=out_t,
        scratch_types=[
            pltpu.VMEM((_CHUNK,), jnp.int32),
            pltpu.VMEM((_CHUNK,), jnp.int32),
            pltpu.VMEM((_CHUNK, D), jnp.float32),
            pltpu.VMEM((_CHUNK, D), jnp.float32),
            pltpu.VMEM_SHARED((npad, D), jnp.float32),
            pltpu.SemaphoreType.DMA,
            pltpu.SemaphoreType.DMA,
        ],
    )
    def k(nodes_h, s_h, r_h, inc_h, outg_h, ix_s, ix_r, rw_s, rw_r, tbl,
          sem_s, sem_r):
        cid = lax.axis_index("c")
        sid = lax.axis_index("s")
        wid = sid * nc + cid
        base = wid * per_w
        row0 = sid * rt

        for coff, csz in stage_chunks:
            pltpu.sync_copy(nodes_h.at[pl.ds(row0 + coff, csz)],
                            rw_s.at[pl.ds(0, csz)])
            pltpu.sync_copy(rw_s.at[pl.ds(0, csz)],
                            tbl.at[pl.ds(row0 + coff, csz)])
        plsc.subcore_barrier()

        def body(i, _):
            off = base + i * _CHUNK
            pltpu.sync_copy(s_h.at[pl.ds(off, _CHUNK)], ix_s)
            pltpu.sync_copy(r_h.at[pl.ds(off, _CHUNK)], ix_r)
            a = pltpu.async_copy(tbl.at[ix_s], rw_s, sem_s)
            b = pltpu.async_copy(tbl.at[ix_r], rw_r, sem_r)
            a.wait()
            pltpu.sync_copy(rw_s, inc_h.at[pl.ds(off, _CHUNK)])
            b.wait()
            pltpu.sync_copy(rw_r, outg_h.at[pl.ds(off, _CHUNK)])
            return 0

        lax.fori_loop(0, n_ch, body, 0, unroll=False)

    return k(nodes, s_idx, r_idx)


def _sc_scatter(msg, r_idx, nrow):
    nc, ns = _sc_counts()
    nw = nc * ns
    ep = msg.shape[0]
    per_w = ep // nw
    n_ch = per_w // _CHUNK
    rows_t = nrow // ns
    mesh = plsc.VectorSubcoreMesh(core_axis_name="c", subcore_axis_name="s")
    out_t = jax.ShapeDtypeStruct((nc, nrow, D), jnp.float32)

    chunks = _chunks_of(rows_t, _CHUNK)

    @functools.partial(
        pl.kernel, mesh=mesh, out_type=out_t,
        scratch_types=[
            pltpu.VMEM((_CHUNK,), jnp.int32),
            pltpu.VMEM((_CHUNK, D), jnp.float32),
            pltpu.VMEM_SHARED((nrow, D), jnp.float32),
            pltpu.SemaphoreType.DMA,
        ],
    )
    def k(msg_h, r_h, out_h, ix, rw, acc, sem):
        cid = lax.axis_index("c")
        sid = lax.axis_index("s")
        wid = sid * nc + cid
        base = wid * per_w
        row0 = sid * rows_t

        def zr(r, _):
            def zc(c, __):
                rw[r, pl.ds(c * 16, 16)] = jnp.zeros((16,), jnp.float32)
                return 0
            lax.fori_loop(0, D // 16, zc, 0, unroll=True)
            return 0

        lax.fori_loop(0, _CHUNK, zr, 0, unroll=False)
        for coff, csz in chunks:
            pltpu.sync_copy(rw.at[pl.ds(0, csz)],
                            acc.at[pl.ds(row0 + coff, csz)])
        plsc.subcore_barrier()

        def body(i, _):
            off = base + i * _CHUNK
            pltpu.sync_copy(r_h.at[pl.ds(off, _CHUNK)], ix)
            pltpu.sync_copy(msg_h.at[pl.ds(off, _CHUNK)], rw)
            pltpu.sync_copy(rw, acc.at[ix], add=True)
            return 0

        lax.fori_loop(0, n_ch, body, 0, unroll=False)
        plsc.subcore_barrier()

        for coff, csz in chunks:
            pltpu.sync_copy(acc.at[pl.ds(row0 + coff, csz)],
                            rw.at[pl.ds(0, csz)])
            pltpu.sync_copy(rw.at[pl.ds(0, csz)],
                            out_h.at[cid, pl.ds(row0 + coff, csz)])

    return k(msg, r_idx)


def kernel(x, node_attr, edge_attr, edge_index, graph_idx, W_embed, W_msg0,
           W_msg1, W_upd0, W_upd1, W_pre0, W_pre1, W_post0, W_out):
    n, d = x.shape
    e = edge_index.shape[1]
    nc, ns = _sc_counts()
    nw = nc * ns
    quant = nw * _CHUNK
    ep = ((e + quant - 1) // quant) * quant
    pad = ep - e
    npad = ((n + 1 + 127) // 128) * 128
    np_ = npad - n

    senders = edge_index[0].astype(jnp.int32)
    receivers = edge_index[1].astype(jnp.int32)
    zpad = jnp.zeros((pad,), jnp.int32)
    s_p = jnp.concatenate([senders, zpad])
    r_p = jnp.concatenate([receivers, zpad])
    r_scat = jnp.concatenate([receivers, jnp.full((pad,), n, jnp.int32)])
    ea_p = jnp.concatenate(
        [edge_attr, jnp.zeros((pad, A), jnp.float32)], axis=0)
    x_p = jnp.concatenate([x, jnp.zeros((np_, d), jnp.float32)])
    nattr_p = jnp.concatenate([node_attr, jnp.zeros((np_, A), jnp.float32)])

    w_embed_r = W_embed.reshape(D, A * D)
    nodes = _embed(x_p, nattr_p, w_embed_r)

    num_layers = W_msg0.shape[0]
    for l in range(num_layers):
        w0a = W_msg0[l, :D].reshape(D, A * D)
        w0b = W_msg0[l, D:].reshape(D, A * D)
        w1 = W_msg1[l].reshape(D, A * D)
        u0a = W_upd0[l, :D].reshape(D, A * D)
        u0b = W_upd0[l, D:].reshape(D, A * D)
        u1 = W_upd1[l].reshape(D, A * D)

        inc, outg = _sc_gather(nodes, s_p, r_p)
        msg = _edge_mlp(inc, outg, ea_p, w0a, w0b, w1)
        agg = _sc_scatter(msg, r_scat, npad)
        nodes = _update(nodes, agg, nattr_p, u0a, u0b, u1)

    h = _prepool(nodes, nattr_p, W_pre0.reshape(D, A * D),
                 W_pre1.reshape(D, A * D))
    gi_p = jnp.concatenate(
        [graph_idx.astype(jnp.int32), jnp.full((np_,), G, jnp.int32)])
    out = _pool_decode(h, gi_p.reshape(npad, 1), W_post0, W_out)
    return out.reshape(G)


# speedup vs baseline: 3.4486x; 1.1305x over previous
import functools
import math

import jax
import jax.numpy as jnp
from jax import lax
from jax.experimental import pallas as pl
from jax.experimental.pallas import tpu as pltpu
from jax.experimental.pallas import tpu_sc as plsc

D = 128
A = 4
G = 16
_CHUNK = 128


def _silu(v):
    return v * jax.nn.sigmoid(v)


def _contract(y, ea, scale):
    acc = ea[:, 0:1] * y[:, 0:D]
    for j in range(1, A):
        acc = acc + ea[:, j:j + 1] * y[:, j * D:(j + 1) * D]
    return acc * scale


def _sc_counts():
    try:
        info = plsc.get_sparse_core_info()
        return int(info.num_cores), int(info.num_subcores)
    except Exception:
        return 2, 16


def _embed_body(x_ref, a_ref, w_ref, o_ref):
    y = jnp.dot(x_ref[...], w_ref[...], preferred_element_type=jnp.float32)
    o_ref[...] = _contract(y, a_ref[...], 1.0 / math.sqrt(D * A))


def _embed(x, nattr, w_r):
    n = x.shape[0]
    bn = n // 16
    return pl.pallas_call(
        _embed_body,
        grid=(n // bn,),
        in_specs=[
            pl.BlockSpec((bn, D), lambda i: (i, 0)),
            pl.BlockSpec((bn, A), lambda i: (i, 0)),
            pl.BlockSpec((D, A * D), lambda i: (0, 0)),
        ],
        out_specs=pl.BlockSpec((bn, D), lambda i: (i, 0)),
        out_shape=jax.ShapeDtypeStruct((n, D), jnp.float32),
    )(x, nattr, w_r)


def _edge_body(inc_ref, outg_ref, ea_ref, w0a_ref, w0b_ref, w1_ref, o_ref):
    ea = ea_ref[...]
    y0 = jnp.dot(inc_ref[...], w0a_ref[...], preferred_element_type=jnp.float32)
    y0 = y0 + jnp.dot(outg_ref[...], w0b_ref[...],
                      preferred_element_type=jnp.float32)
    m = _silu(_contract(y0, ea, 1.0 / math.sqrt(2 * D * A)))
    y1 = jnp.dot(m, w1_ref[...], preferred_element_type=jnp.float32)
    o_ref[...] = _silu(_contract(y1, ea, 1.0 / math.sqrt(D * A)))


def _edge_mlp(inc, outg, ea, w0a, w0b, w1):
    ep = inc.shape[0]
    be = 2048
    return pl.pallas_call(
        _edge_body,
        grid=(ep // be,),
        in_specs=[
            pl.BlockSpec((be, D), lambda i: (i, 0)),
            pl.BlockSpec((be, D), lambda i: (i, 0)),
            pl.BlockSpec((be, A), lambda i: (i, 0)),
            pl.BlockSpec((D, A * D), lambda i: (0, 0)),
            pl.BlockSpec((D, A * D), lambda i: (0, 0)),
            pl.BlockSpec((D, A * D), lambda i: (0, 0)),
        ],
        out_specs=pl.BlockSpec((be, D), lambda i: (i, 0)),
        out_shape=jax.ShapeDtypeStruct((ep, D), jnp.float32),
    )(inc, outg, ea, w0a, w0b, w1)


def _update_body(nd_ref, a0_ref, a1_ref, na_ref, w0a_ref, w0b_ref, w1_ref,
                 o_ref):
    nd = nd_ref[...]
    agg = a0_ref[0] + a1_ref[0]
    na = na_ref[...]
    y0 = jnp.dot(nd, w0a_ref[...], preferred_element_type=jnp.float32)
    y0 = y0 + jnp.dot(agg, w0b_ref[...], preferred_element_type=jnp.float32)
    u = _silu(_contract(y0, na, 1.0 / math.sqrt(2 * D * A)))
    y1 = jnp.dot(u, w1_ref[...], preferred_element_type=jnp.float32)
    o_ref[...] = nd + _contract(y1, na, 1.0 / math.sqrt(D * A))


def _update(nodes, agg, nattr, w0a, w0b, w1):
    n = nodes.shape[0]
    bn = n // 16
    return pl.pallas_call(
        _update_body,
        grid=(n // bn,),
        in_specs=[
            pl.BlockSpec((bn, D), lambda i: (i, 0)),
            pl.BlockSpec((1, bn, D), lambda i: (0, i, 0)),
            pl.BlockSpec((1, bn, D), lambda i: (1, i, 0)),
            pl.BlockSpec((bn, A), lambda i: (i, 0)),
            pl.BlockSpec((D, A * D), lambda i: (0, 0)),
            pl.BlockSpec((D, A * D), lambda i: (0, 0)),
            pl.BlockSpec((D, A * D), lambda i: (0, 0)),
        ],
        out_specs=pl.BlockSpec((bn, D), lambda i: (i, 0)),
        out_shape=jax.ShapeDtypeStruct((n, D), jnp.float32),
    )(nodes, agg, agg, nattr, w0a, w0b, w1)


def _prepool_body(nd_ref, na_ref, w0_ref, w1_ref, o_ref):
    na = na_ref[...]
    y0 = jnp.dot(nd_ref[...], w0_ref[...], preferred_element_type=jnp.float32)
    h = _silu(_contract(y0, na, 1.0 / math.sqrt(D * A)))
    y1 = jnp.dot(h, w1_ref[...], preferred_element_type=jnp.float32)
    o_ref[...] = _contract(y1, na, 1.0 / math.sqrt(D * A))


def _prepool(nodes, nattr, w0, w1):
    n = nodes.shape[0]
    bn = n // 16
    return pl.pallas_call(
        _prepool_body,
        grid=(n // bn,),
        in_specs=[
            pl.BlockSpec((bn, D), lambda i: (i, 0)),
            pl.BlockSpec((bn, A), lambda i: (i, 0)),
            pl.BlockSpec((D, A * D), lambda i: (0, 0)),
            pl.BlockSpec((D, A * D), lambda i: (0, 0)),
        ],
        out_specs=pl.BlockSpec((bn, D), lambda i: (i, 0)),
        out_shape=jax.ShapeDtypeStruct((n, D), jnp.float32),
    )(nodes, nattr, w0, w1)


def _pool_body(h_ref, gi_ref, wpost_ref, wout_ref, o_ref, sums, cnt):
    i = pl.program_id(0)

    @pl.when(i == 0)
    def _():
        sums[...] = jnp.zeros_like(sums)
        cnt[...] = jnp.zeros_like(cnt)

    gi = gi_ref[...]
    bn = gi.shape[0]
    m = (gi == lax.broadcasted_iota(jnp.int32, (bn, G), 1)).astype(jnp.float32)
    h = h_ref[...]
    dn = (((0,), (0,)), ((), ()))
    sums[...] += lax.dot_general(m, h, dn, preferred_element_type=jnp.float32)
    cnt[...] += lax.dot_general(m, jnp.ones_like(h), dn,
                                preferred_element_type=jnp.float32)
    pooled = sums[...] / jnp.maximum(cnt[...], 1.0)
    h2 = _silu(jnp.dot(pooled, wpost_ref[...],
                       preferred_element_type=jnp.float32) / math.sqrt(D))
    o_ref[...] = jnp.dot(h2, wout_ref[...],
                         preferred_element_type=jnp.float32) / math.sqrt(D)


def _pool_decode(h, gi2d, wpost, wout):
    n = h.shape[0]
    bn = n // 16
    return pl.pallas_call(
        _pool_body,
        grid=(n // bn,),
        in_specs=[
            pl.BlockSpec((bn, D), lambda i: (i, 0)),
            pl.BlockSpec((bn, 1), lambda i: (i, 0)),
            pl.BlockSpec((D, D), lambda i: (0, 0)),
            pl.BlockSpec((D, 1), lambda i: (0, 0)),
        ],
        out_specs=pl.BlockSpec((G, 1), lambda i: (0, 0)),
        out_shape=jax.ShapeDtypeStruct((G, 1), jnp.float32),
        scratch_shapes=[
            pltpu.VMEM((G, D), jnp.float32),
            pltpu.VMEM((G, D), jnp.float32),
        ],
    )(h, gi2d, wpost, wout)


def _chunks_of(total, cap):
    out, off = [], 0
    while off < total:
        sz = min(cap, total - off)
        out.append((off, sz))
        off += sz
    return out


def _sc_gather(nodes, s_idx, r_idx):
    nc, ns = _sc_counts()
    nw = nc * ns
    ep = s_idx.shape[0]
    npad = nodes.shape[0]
    rt = npad // ns
    per_w = ep // nw
    ch = 64
    n2 = per_w // (2 * ch)
    stage_chunks = _chunks_of(rt, ch)
    mesh = plsc.VectorSubcoreMesh(core_axis_name="c", subcore_axis_name="s")
    out_t = (jax.ShapeDtypeStruct((ep, D), jnp.float32),
             jax.ShapeDtypeStruct((ep, D), jnp.float32))

    @functools.partial(
        pl.kernel, mesh=mesh, out_type=out_t,
        scratch_types=[
            pltpu.VMEM((per_w,), jnp.int32),
            pltpu.VMEM((per_w,), jnp.int32),
            pltpu.VMEM((2, ch, D), jnp.float32),
            pltpu.VMEM((2, ch, D), jnp.float32),
            pltpu.VMEM_SHARED((npad, D), jnp.float32),
            [pltpu.SemaphoreType.DMA] * 4,
            [pltpu.SemaphoreType.DMA] * 4,
        ],
    )
    def k(nodes_h, s_h, r_h, inc_h, outg_h, ix_s, ix_r, rw_s, rw_r, tbl,
          sg, sw):
        cid = lax.axis_index("c")
        sid = lax.axis_index("s")
        wid = sid * nc + cid
        base = wid * per_w
        row0 = sid * rt

        for coff, csz in stage_chunks:
            pltpu.sync_copy(nodes_h.at[pl.ds(row0 + coff, csz)],
                            rw_s.at[0, pl.ds(0, csz)])
            pltpu.sync_copy(rw_s.at[0, pl.ds(0, csz)],
                            tbl.at[pl.ds(row0 + coff, csz)])
        pltpu.sync_copy(s_h.at[pl.ds(base, per_w)], ix_s)
        pltpu.sync_copy(r_h.at[pl.ds(base, per_w)], ix_r)
        plsc.subcore_barrier()

        def gath(c, b):
            return (pltpu.async_copy(tbl.at[ix_s.at[pl.ds(c * ch, ch)]],
                                     rw_s.at[b], sg[b]),
                    pltpu.async_copy(tbl.at[ix_r.at[pl.ds(c * ch, ch)]],
                                     rw_r.at[b], sg[2 + b]))

        def wait_gath(c, b):
            pltpu.make_async_copy(tbl.at[ix_s.at[pl.ds(c * ch, ch)]],
                                  rw_s.at[b], sg[b]).wait()
            pltpu.make_async_copy(tbl.at[ix_r.at[pl.ds(c * ch, ch)]],
                                  rw_r.at[b], sg[2 + b]).wait()

        def write(c, b):
            off = base + c * ch
            return (pltpu.async_copy(rw_s.at[b], inc_h.at[pl.ds(off, ch)],
                                     sw[b]),
                    pltpu.async_copy(rw_r.at[b], outg_h.at[pl.ds(off, ch)],
                                     sw[2 + b]))

        def wait_write(c, b):
            off = base + c * ch
            pltpu.make_async_copy(rw_s.at[b], inc_h.at[pl.ds(off, ch)],
                                  sw[b]).wait()
            pltpu.make_async_copy(rw_r.at[b], outg_h.at[pl.ds(off, ch)],
                                  sw[2 + b]).wait()

        gath(0, 0)

        def body(j, _):
            c0 = 2 * j
            @pl.when(j > 0)
            def _():
                wait_write(c0 - 1, 1)

            gath(c0 + 1, 1)
            wait_gath(c0, 0)
            write(c0, 0)
            wait_gath(c0 + 1, 1)
            write(c0 + 1, 1)
            wait_write(c0, 0)

            @pl.when(j < n2 - 1)
            def _():
                gath(c0 + 2, 0)

            return 0

        lax.fori_loop(0, n2, body, 0, unroll=False)
        wait_write(2 * n2 - 1, 1)

    return k(nodes, s_idx, r_idx)


def _sc_scatter(msg, r_idx, nrow):
    nc, ns = _sc_counts()
    nw = nc * ns
    ep = msg.shape[0]
    per_w = ep // nw
    n_ch = per_w // _CHUNK
    rows_t = nrow // ns
    mesh = plsc.VectorSubcoreMesh(core_axis_name="c", subcore_axis_name="s")
    out_t = jax.ShapeDtypeStruct((nc, nrow, D), jnp.float32)

    chunks = _chunks_of(rows_t, _CHUNK)

    @functools.partial(
        pl.kernel, mesh=mesh, out_type=out_t,
        scratch_types=[
            pltpu.VMEM((_CHUNK,), jnp.int32),
            pltpu.VMEM((_CHUNK, D), jnp.float32),
            pltpu.VMEM_SHARED((nrow, D), jnp.float32),
            pltpu.SemaphoreType.DMA,
        ],
    )
    def k(msg_h, r_h, out_h, ix, rw, acc, sem):
        cid = lax.axis_index("c")
        sid = lax.axis_index("s")
        wid = sid * nc + cid
        base = wid * per_w
        row0 = sid * rows_t

        def zr(r, _):
            def zc(c, __):
                rw[r, pl.ds(c * 16, 16)] = jnp.zeros((16,), jnp.float32)
                return 0
            lax.fori_loop(0, D // 16, zc, 0, unroll=True)
            return 0

        lax.fori_loop(0, _CHUNK, zr, 0, unroll=False)
        for coff, csz in chunks:
            pltpu.sync_copy(rw.at[pl.ds(0, csz)],
                            acc.at[pl.ds(row0 + coff, csz)])
        plsc.subcore_barrier()

        def body(i, _):
            off = base + i * _CHUNK
            pltpu.sync_copy(r_h.at[pl.ds(off, _CHUNK)], ix)
            pltpu.sync_copy(msg_h.at[pl.ds(off, _CHUNK)], rw)
            pltpu.sync_copy(rw, acc.at[ix], add=True)
            return 0

        lax.fori_loop(0, n_ch, body, 0, unroll=False)
        plsc.subcore_barrier()

        for coff, csz in chunks:
            pltpu.sync_copy(acc.at[pl.ds(row0 + coff, csz)],
                            rw.at[pl.ds(0, csz)])
            pltpu.sync_copy(rw.at[pl.ds(0, csz)],
                            out_h.at[cid, pl.ds(row0 + coff, csz)])

    return k(msg, r_idx)


def kernel(x, node_attr, edge_attr, edge_index, graph_idx, W_embed, W_msg0,
           W_msg1, W_upd0, W_upd1, W_pre0, W_pre1, W_post0, W_out):
    n, d = x.shape
    e = edge_index.shape[1]
    nc, ns = _sc_counts()
    nw = nc * ns
    quant = nw * _CHUNK
    ep = ((e + quant - 1) // quant) * quant
    pad = ep - e
    npad = ((n + 1 + 127) // 128) * 128
    np_ = npad - n

    senders = edge_index[0].astype(jnp.int32)
    receivers = edge_index[1].astype(jnp.int32)
    zpad = jnp.zeros((pad,), jnp.int32)
    s_p = jnp.concatenate([senders, zpad])
    r_p = jnp.concatenate([receivers, zpad])
    r_scat = jnp.concatenate([receivers, jnp.full((pad,), n, jnp.int32)])
    ea_p = jnp.concatenate(
        [edge_attr, jnp.zeros((pad, A), jnp.float32)], axis=0)
    x_p = jnp.concatenate([x, jnp.zeros((np_, d), jnp.float32)])
    nattr_p = jnp.concatenate([node_attr, jnp.zeros((np_, A), jnp.float32)])

    w_embed_r = W_embed.reshape(D, A * D)
    nodes = _embed(x_p, nattr_p, w_embed_r)

    num_layers = W_msg0.shape[0]
    for l in range(num_layers):
        w0a = W_msg0[l, :D].reshape(D, A * D)
        w0b = W_msg0[l, D:].reshape(D, A * D)
        w1 = W_msg1[l].reshape(D, A * D)
        u0a = W_upd0[l, :D].reshape(D, A * D)
        u0b = W_upd0[l, D:].reshape(D, A * D)
        u1 = W_upd1[l].reshape(D, A * D)

        inc, outg = _sc_gather(nodes, s_p, r_p)
        msg = _edge_mlp(inc, outg, ea_p, w0a, w0b, w1)
        agg = _sc_scatter(msg, r_scat, npad)
        nodes = _update(nodes, agg, nattr_p, u0a, u0b, u1)

    h = _prepool(nodes, nattr_p, W_pre0.reshape(D, A * D),
                 W_pre1.reshape(D, A * D))
    gi_p = jnp.concatenate(
        [graph_idx.astype(jnp.int32), jnp.full((np_,), G, jnp.int32)])
    out = _pool_decode(h, gi_p.reshape(npad, 1), W_post0, W_out)
    return out.reshape(G)
